# Initial kernel scaffold; baseline (speedup 1.0000x reference)
import jax
import jax.numpy as jnp
from jax.experimental import pallas as pl


def kernel(x, lora_a, lora_b, slot_ids):
    raise NotImplementedError("write your pallas kernel here")


# speedup vs baseline: 4.6536x; 4.6536x over previous
import jax
import jax.numpy as jnp
from jax.experimental import pallas as pl
from jax.experimental.pallas import tpu as pltpu

_NUM_SLOTS = 8
_RANK = 64
_TOKENS = 4096
_D_IN = 2048
_D_OUT = 4096
_BT = 256


def _lora_block_kernel(slot_smem, x_ref, slots_ref, a_ref, b_ref, o_ref):
    i = pl.program_id(0)
    e_lo = slot_smem[i * _BT]
    e_hi = slot_smem[i * _BT + _BT - 1]
    x = x_ref[...]
    slots = slots_ref[...]

    def body(e, acc):
        inter = jnp.dot(x, a_ref[e], preferred_element_type=jnp.float32)
        out_e = jnp.dot(inter, b_ref[e], preferred_element_type=jnp.float32)
        mask = (slots == e).astype(jnp.float32)
        return acc + out_e * mask

    acc = jax.lax.fori_loop(
        e_lo, e_hi + 1, body, jnp.zeros((_BT, _D_OUT), jnp.float32)
    )
    o_ref[...] = acc


def kernel(x, lora_a, lora_b, slot_ids):
    slot_ids = slot_ids.astype(jnp.int32)
    slots2d = slot_ids.reshape(_TOKENS, 1)
    grid_spec = pltpu.PrefetchScalarGridSpec(
        num_scalar_prefetch=1,
        grid=(_TOKENS // _BT,),
        in_specs=[
            pl.BlockSpec((_BT, _D_IN), lambda i, s: (i, 0)),
            pl.BlockSpec((_BT, 1), lambda i, s: (i, 0)),
            pl.BlockSpec((_NUM_SLOTS, _D_IN, _RANK), lambda i, s: (0, 0, 0)),
            pl.BlockSpec((_NUM_SLOTS, _RANK, _D_OUT), lambda i, s: (0, 0, 0)),
        ],
        out_specs=pl.BlockSpec((_BT, _D_OUT), lambda i, s: (i, 0)),
    )
    return pl.pallas_call(
        _lora_block_kernel,
        grid_spec=grid_spec,
        out_shape=jax.ShapeDtypeStruct((_TOKENS, _D_OUT), jnp.float32),
    )(slot_ids, x, slots2d, lora_a, lora_b)
